# baseline (speedup 1.0000x reference)
import jax
import jax.numpy as jnp
from jax.experimental import pallas as pl
from jax.experimental.pallas import tpu as pltpu

_T = 4
_H = 128
_ROWS = 1000


def _lstm_block_kernel(x_ref,
                       wxi_ref, whi_ref, wxf_ref, whf_ref,
                       wxc_ref, whc_ref, wxo_ref, who_ref,
                       bxi_ref, bhi_ref, bi_ref,
                       bxf_ref, bhf_ref, bf_ref,
                       bxc_ref, bhc_ref, bc_ref,
                       bxo_ref, bho_ref, bo_ref,
                       wci_ref, wcf_ref, wco_ref,
                       wout_ref, bout_ref,
                       out_ref, h_ref, c_ref):
    r = h_ref.shape[0]
    bias_i = bxi_ref[...] + bhi_ref[...] + bi_ref[...]
    bias_f = bxf_ref[...] + bhf_ref[...] + bf_ref[...]
    bias_c = bxc_ref[...] + bhc_ref[...] + bc_ref[...]
    bias_o = bxo_ref[...] + bho_ref[...] + bo_ref[...]
    wci = wci_ref[...]
    wcf = wcf_ref[...]
    wco = wco_ref[...]

    h = jnp.zeros((r, _H), jnp.float32)
    c = jnp.zeros((r, _H), jnp.float32)
    for t in range(_T):
        xt = x_ref[t]
        zi = (jnp.dot(xt, wxi_ref[...], preferred_element_type=jnp.float32)
              + jnp.dot(h, whi_ref[...], preferred_element_type=jnp.float32)
              + bias_i)
        zf = (jnp.dot(xt, wxf_ref[...], preferred_element_type=jnp.float32)
              + jnp.dot(h, whf_ref[...], preferred_element_type=jnp.float32)
              + bias_f)
        zc = (jnp.dot(xt, wxc_ref[...], preferred_element_type=jnp.float32)
              + jnp.dot(h, whc_ref[...], preferred_element_type=jnp.float32)
              + bias_c)
        zo = (jnp.dot(xt, wxo_ref[...], preferred_element_type=jnp.float32)
              + jnp.dot(h, who_ref[...], preferred_element_type=jnp.float32)
              + bias_o)
        gate_i = jax.nn.sigmoid(zi + wci * c)
        gate_f = jax.nn.sigmoid(zf + wcf * c)
        cand = jnp.tanh(zc)
        c = gate_f * c + gate_i * cand
        gate_o = jax.nn.sigmoid(zo + wco * c)
        h = gate_o * jnp.tanh(c)

    h_ref[...] = h
    c_ref[...] = c
    out_ref[...] = (jnp.dot(h, wout_ref[...], preferred_element_type=jnp.float32)
                    + bout_ref[...])


def kernel(x, edge_index, edge_weight, Wl, bl, Wr, br, att, gat_bias,
           Wxi, bxi, Whi, bhi, Wxf, bxf, Whf, bhf, Wxc, bxc, Whc, bhc,
           Wxo, bxo, Who, bho, wci, wcf, wco, bi, bf, bc, bo, Wout, bout):
    t_win, n, f = x.shape
    assert t_win == _T and f == _H
    rows = _ROWS
    grid = n // rows

    mat = pl.BlockSpec((_H, _H), lambda i: (0, 0))
    vec = pl.BlockSpec((_H,), lambda i: (0,))

    out2d, h, c = pl.pallas_call(
        _lstm_block_kernel,
        grid=(grid,),
        in_specs=[
            pl.BlockSpec((_T, rows, _H), lambda i: (0, i, 0)),
            mat, mat, mat, mat, mat, mat, mat, mat,
            vec, vec, vec, vec, vec, vec, vec, vec, vec, vec, vec, vec,
            vec, vec, vec,
            pl.BlockSpec((_H, 1), lambda i: (0, 0)),
            pl.BlockSpec((1,), lambda i: (0,)),
        ],
        out_specs=[
            pl.BlockSpec((rows, 1), lambda i: (i, 0)),
            pl.BlockSpec((rows, _H), lambda i: (i, 0)),
            pl.BlockSpec((rows, _H), lambda i: (i, 0)),
        ],
        out_shape=[
            jax.ShapeDtypeStruct((n, 1), jnp.float32),
            jax.ShapeDtypeStruct((n, _H), jnp.float32),
            jax.ShapeDtypeStruct((n, _H), jnp.float32),
        ],
        compiler_params=pltpu.CompilerParams(
            dimension_semantics=("parallel",)),
    )(x, Wxi, Whi, Wxf, Whf, Wxc, Whc, Wxo, Who,
      bxi, bhi, bi, bxf, bhf, bf, bxc, bhc, bc, bxo, bho, bo,
      wci, wcf, wco, Wout, bout)

    return (out2d[:, 0], (h, c))


# speedup vs baseline: 1.2696x; 1.2696x over previous
import jax
import jax.numpy as jnp
from jax.experimental import pallas as pl
from jax.experimental.pallas import tpu as pltpu

_T = 4
_H = 128
_ROWS = 1000


def _lstm_block_kernel(x_ref,
                       wxi_ref, whi_ref, wxf_ref, whf_ref,
                       wxc_ref, whc_ref, wxo_ref, who_ref,
                       bxi_ref, bhi_ref, bi_ref,
                       bxf_ref, bhf_ref, bf_ref,
                       bxc_ref, bhc_ref, bc_ref,
                       bxo_ref, bho_ref, bo_ref,
                       wci_ref, wcf_ref, wco_ref,
                       wout_ref, bout_ref,
                       out_ref, h_ref, c_ref):
    r = h_ref.shape[0]
    wx = jnp.concatenate(
        [wxi_ref[...], wxf_ref[...], wxc_ref[...], wxo_ref[...]], axis=1)
    wh = jnp.concatenate(
        [whi_ref[...], whf_ref[...], whc_ref[...], who_ref[...]], axis=1)
    bias_i = bxi_ref[...] + bhi_ref[...] + bi_ref[...]
    bias_f = bxf_ref[...] + bhf_ref[...] + bf_ref[...]
    bias_c = bxc_ref[...] + bhc_ref[...] + bc_ref[...]
    bias_o = bxo_ref[...] + bho_ref[...] + bo_ref[...]
    wci = wci_ref[...]
    wcf = wcf_ref[...]
    wco = wco_ref[...]

    x_all = x_ref[...].reshape(_T * r, _H)
    xw_all = jnp.dot(x_all, wx, preferred_element_type=jnp.float32)

    h = jnp.zeros((r, _H), jnp.float32)
    c = jnp.zeros((r, _H), jnp.float32)
    for t in range(_T):
        g = xw_all[t * r:(t + 1) * r, :] + jnp.dot(
            h, wh, preferred_element_type=jnp.float32)
        gate_i = jax.nn.sigmoid(g[:, 0:_H] + (bias_i + wci * c))
        gate_f = jax.nn.sigmoid(g[:, _H:2 * _H] + (bias_f + wcf * c))
        cand = jnp.tanh(g[:, 2 * _H:3 * _H] + bias_c)
        c = gate_f * c + gate_i * cand
        gate_o = jax.nn.sigmoid(g[:, 3 * _H:4 * _H] + (bias_o + wco * c))
        h = gate_o * jnp.tanh(c)

    h_ref[...] = h
    c_ref[...] = c
    out_ref[...] = (jnp.dot(h, wout_ref[...], preferred_element_type=jnp.float32)
                    + bout_ref[...])


def kernel(x, edge_index, edge_weight, Wl, bl, Wr, br, att, gat_bias,
           Wxi, bxi, Whi, bhi, Wxf, bxf, Whf, bhf, Wxc, bxc, Whc, bhc,
           Wxo, bxo, Who, bho, wci, wcf, wco, bi, bf, bc, bo, Wout, bout):
    t_win, n, f = x.shape
    assert t_win == _T and f == _H
    rows = _ROWS
    grid = n // rows

    mat = pl.BlockSpec((_H, _H), lambda i: (0, 0))
    vec = pl.BlockSpec((_H,), lambda i: (0,))

    out2d, h, c = pl.pallas_call(
        _lstm_block_kernel,
        grid=(grid,),
        in_specs=[
            pl.BlockSpec((_T, rows, _H), lambda i: (0, i, 0)),
            mat, mat, mat, mat, mat, mat, mat, mat,
            vec, vec, vec, vec, vec, vec, vec, vec, vec, vec, vec, vec,
            vec, vec, vec,
            pl.BlockSpec((_H, 1), lambda i: (0, 0)),
            pl.BlockSpec((1,), lambda i: (0,)),
        ],
        out_specs=[
            pl.BlockSpec((rows, 1), lambda i: (i, 0)),
            pl.BlockSpec((rows, _H), lambda i: (i, 0)),
            pl.BlockSpec((rows, _H), lambda i: (i, 0)),
        ],
        out_shape=[
            jax.ShapeDtypeStruct((n, 1), jnp.float32),
            jax.ShapeDtypeStruct((n, _H), jnp.float32),
            jax.ShapeDtypeStruct((n, _H), jnp.float32),
        ],
        compiler_params=pltpu.CompilerParams(
            dimension_semantics=("parallel",)),
    )(x, Wxi, Whi, Wxf, Whf, Wxc, Whc, Wxo, Who,
      bxi, bhi, bi, bxf, bhf, bf, bxc, bhc, bc, bxo, bho, bo,
      wci, wcf, wco, Wout, bout)

    return (out2d[:, 0], (h, c))


# speedup vs baseline: 1.2857x; 1.0126x over previous
import jax
import jax.numpy as jnp
from jax.experimental import pallas as pl
from jax.experimental.pallas import tpu as pltpu

_T = 4
_H = 128
_ROWS = 1000


def _sigmoid(z):
    return 0.5 * jnp.tanh(0.5 * z) + 0.5


def _lstm_block_kernel(x_ref,
                       wxi_ref, whi_ref, wxf_ref, whf_ref,
                       wxc_ref, whc_ref, wxo_ref, who_ref,
                       bxi_ref, bhi_ref, bi_ref,
                       bxf_ref, bhf_ref, bf_ref,
                       bxc_ref, bhc_ref, bc_ref,
                       bxo_ref, bho_ref, bo_ref,
                       wci_ref, wcf_ref, wco_ref,
                       wout_ref, bout_ref,
                       out_ref, h_ref, c_ref):
    r = h_ref.shape[0]
    wx = jnp.concatenate(
        [wxi_ref[...], wxf_ref[...], wxc_ref[...], wxo_ref[...]], axis=1)
    wh = jnp.concatenate(
        [whi_ref[...], whf_ref[...], whc_ref[...], who_ref[...]], axis=1)
    bias_i = bxi_ref[...] + bhi_ref[...] + bi_ref[...]
    bias_f = bxf_ref[...] + bhf_ref[...] + bf_ref[...]
    bias_c = bxc_ref[...] + bhc_ref[...] + bc_ref[...]
    bias_o = bxo_ref[...] + bho_ref[...] + bo_ref[...]
    wci = wci_ref[...]
    wcf = wcf_ref[...]
    wco = wco_ref[...]

    x_all = x_ref[...].reshape(_T * r, _H)
    xw_all = jnp.dot(x_all, wx, preferred_element_type=jnp.float32)

    h = jnp.zeros((r, _H), jnp.float32)
    c = jnp.zeros((r, _H), jnp.float32)
    for t in range(_T):
        g = xw_all[t * r:(t + 1) * r, :] + jnp.dot(
            h, wh, preferred_element_type=jnp.float32)
        gate_i = _sigmoid(g[:, 0:_H] + (bias_i + wci * c))
        gate_f = _sigmoid(g[:, _H:2 * _H] + (bias_f + wcf * c))
        cand = jnp.tanh(g[:, 2 * _H:3 * _H] + bias_c)
        c = gate_f * c + gate_i * cand
        gate_o = _sigmoid(g[:, 3 * _H:4 * _H] + (bias_o + wco * c))
        h = gate_o * jnp.tanh(c)

    h_ref[...] = h
    c_ref[...] = c
    out_ref[...] = (jnp.dot(h, wout_ref[...], preferred_element_type=jnp.float32)
                    + bout_ref[...])


def kernel(x, edge_index, edge_weight, Wl, bl, Wr, br, att, gat_bias,
           Wxi, bxi, Whi, bhi, Wxf, bxf, Whf, bhf, Wxc, bxc, Whc, bhc,
           Wxo, bxo, Who, bho, wci, wcf, wco, bi, bf, bc, bo, Wout, bout):
    t_win, n, f = x.shape
    assert t_win == _T and f == _H
    rows = _ROWS
    grid = n // rows

    mat = pl.BlockSpec((_H, _H), lambda i: (0, 0))
    vec = pl.BlockSpec((_H,), lambda i: (0,))

    out2d, h, c = pl.pallas_call(
        _lstm_block_kernel,
        grid=(grid,),
        in_specs=[
            pl.BlockSpec((_T, rows, _H), lambda i: (0, i, 0)),
            mat, mat, mat, mat, mat, mat, mat, mat,
            vec, vec, vec, vec, vec, vec, vec, vec, vec, vec, vec, vec,
            vec, vec, vec,
            pl.BlockSpec((_H, 1), lambda i: (0, 0)),
            pl.BlockSpec((1,), lambda i: (0,)),
        ],
        out_specs=[
            pl.BlockSpec((rows, 1), lambda i: (i, 0)),
            pl.BlockSpec((rows, _H), lambda i: (i, 0)),
            pl.BlockSpec((rows, _H), lambda i: (i, 0)),
        ],
        out_shape=[
            jax.ShapeDtypeStruct((n, 1), jnp.float32),
            jax.ShapeDtypeStruct((n, _H), jnp.float32),
            jax.ShapeDtypeStruct((n, _H), jnp.float32),
        ],
        compiler_params=pltpu.CompilerParams(
            dimension_semantics=("parallel",)),
    )(x, Wxi, Whi, Wxf, Whf, Wxc, Whc, Wxo, Who,
      bxi, bhi, bi, bxf, bhf, bf, bxc, bhc, bc, bxo, bho, bo,
      wci, wcf, wco, Wout, bout)

    return (out2d[:, 0], (h, c))


# speedup vs baseline: 1.7634x; 1.3716x over previous
import jax
import jax.numpy as jnp
from jax.experimental import pallas as pl
from jax.experimental.pallas import tpu as pltpu

_T = 4
_H = 128
_ROWS = 1000


def _sigmoid(z):
    return 0.5 * jnp.tanh(0.5 * z) + 0.5


def _lstm_block_kernel(x_ref,
                       wxi_ref, whi_ref, wxf_ref, whf_ref,
                       wxc_ref, whc_ref, wxo_ref, who_ref,
                       bxi_ref, bhi_ref, bi_ref,
                       bxf_ref, bhf_ref, bf_ref,
                       bxc_ref, bhc_ref, bc_ref,
                       bxo_ref, bho_ref, bo_ref,
                       wci_ref, wcf_ref, wco_ref,
                       wout_ref, bout_ref,
                       out_ref, h_ref, c_ref):
    r = h_ref.shape[0]
    wx = jnp.concatenate(
        [wxi_ref[...], wxf_ref[...], wxc_ref[...], wxo_ref[...]], axis=1)
    wh = jnp.concatenate(
        [whi_ref[...], whf_ref[...], whc_ref[...], who_ref[...]], axis=1)
    bias_i = bxi_ref[...] + bhi_ref[...] + bi_ref[...]
    bias_f = bxf_ref[...] + bhf_ref[...] + bf_ref[...]
    bias_c = bxc_ref[...] + bhc_ref[...] + bc_ref[...]
    bias_o = bxo_ref[...] + bho_ref[...] + bo_ref[...]
    wci = wci_ref[...]
    wcf = wcf_ref[...]
    wco = wco_ref[...]

    w2 = jnp.concatenate([wx, wh], axis=0)

    g = jnp.dot(x_ref[0], wx, preferred_element_type=jnp.float32)
    gate_i = _sigmoid(g[:, 0:_H] + bias_i)
    cand = jnp.tanh(g[:, 2 * _H:3 * _H] + bias_c)
    c = gate_i * cand
    gate_o = _sigmoid(g[:, 3 * _H:4 * _H] + (bias_o + wco * c))
    h = gate_o * jnp.tanh(c)

    for t in range(1, _T):
        xh = jnp.concatenate([x_ref[t], h], axis=1)
        g = jnp.dot(xh, w2, preferred_element_type=jnp.float32)
        gate_i = _sigmoid(g[:, 0:_H] + (bias_i + wci * c))
        gate_f = _sigmoid(g[:, _H:2 * _H] + (bias_f + wcf * c))
        cand = jnp.tanh(g[:, 2 * _H:3 * _H] + bias_c)
        c = gate_f * c + gate_i * cand
        gate_o = _sigmoid(g[:, 3 * _H:4 * _H] + (bias_o + wco * c))
        h = gate_o * jnp.tanh(c)

    h_ref[...] = h
    c_ref[...] = c
    out_ref[...] = (jnp.dot(h, wout_ref[...], preferred_element_type=jnp.float32)
                    + bout_ref[...])


def kernel(x, edge_index, edge_weight, Wl, bl, Wr, br, att, gat_bias,
           Wxi, bxi, Whi, bhi, Wxf, bxf, Whf, bhf, Wxc, bxc, Whc, bhc,
           Wxo, bxo, Who, bho, wci, wcf, wco, bi, bf, bc, bo, Wout, bout):
    t_win, n, f = x.shape
    assert t_win == _T and f == _H
    rows = _ROWS
    grid = n // rows

    mat = pl.BlockSpec((_H, _H), lambda i: (0, 0))
    vec = pl.BlockSpec((_H,), lambda i: (0,))

    out2d, h, c = pl.pallas_call(
        _lstm_block_kernel,
        grid=(grid,),
        in_specs=[
            pl.BlockSpec((_T, rows, _H), lambda i: (0, i, 0)),
            mat, mat, mat, mat, mat, mat, mat, mat,
            vec, vec, vec, vec, vec, vec, vec, vec, vec, vec, vec, vec,
            vec, vec, vec,
            pl.BlockSpec((_H, 1), lambda i: (0, 0)),
            pl.BlockSpec((1,), lambda i: (0,)),
        ],
        out_specs=[
            pl.BlockSpec((rows, 1), lambda i: (i, 0)),
            pl.BlockSpec((rows, _H), lambda i: (i, 0)),
            pl.BlockSpec((rows, _H), lambda i: (i, 0)),
        ],
        out_shape=[
            jax.ShapeDtypeStruct((n, 1), jnp.float32),
            jax.ShapeDtypeStruct((n, _H), jnp.float32),
            jax.ShapeDtypeStruct((n, _H), jnp.float32),
        ],
        compiler_params=pltpu.CompilerParams(
            dimension_semantics=("parallel",)),
    )(x, Wxi, Whi, Wxf, Whf, Wxc, Whc, Wxo, Who,
      bxi, bhi, bi, bxf, bhf, bf, bxc, bhc, bc, bxo, bho, bo,
      wci, wcf, wco, Wout, bout)

    return (out2d[:, 0], (h, c))


# speedup vs baseline: 1.8360x; 1.0412x over previous
import jax
import jax.numpy as jnp
from jax.experimental import pallas as pl
from jax.experimental.pallas import tpu as pltpu

_T = 4
_H = 128
_ROWS = 1000


def _sigmoid_prescaled(zh):
    return 0.5 * jnp.tanh(zh) + 0.5


def _lstm_block_kernel(x_ref,
                       wxi_ref, whi_ref, wxf_ref, whf_ref,
                       wxc_ref, whc_ref, wxo_ref, who_ref,
                       bxi_ref, bhi_ref, bi_ref,
                       bxf_ref, bhf_ref, bf_ref,
                       bxc_ref, bhc_ref, bc_ref,
                       bxo_ref, bho_ref, bo_ref,
                       wci_ref, wcf_ref, wco_ref,
                       wout_ref, bout_ref,
                       out_ref, h_ref, c_ref):
    r = h_ref.shape[0]
    wx = jnp.concatenate(
        [0.5 * wxi_ref[...], 0.5 * wxf_ref[...],
         wxc_ref[...], 0.5 * wxo_ref[...]], axis=1)
    wh = jnp.concatenate(
        [0.5 * whi_ref[...], 0.5 * whf_ref[...],
         whc_ref[...], 0.5 * who_ref[...]], axis=1)
    bias_i = 0.5 * (bxi_ref[...] + bhi_ref[...] + bi_ref[...])
    bias_f = 0.5 * (bxf_ref[...] + bhf_ref[...] + bf_ref[...])
    bias_c = bxc_ref[...] + bhc_ref[...] + bc_ref[...]
    bias_o = 0.5 * (bxo_ref[...] + bho_ref[...] + bo_ref[...])
    wci = 0.5 * wci_ref[...]
    wcf = 0.5 * wcf_ref[...]
    wco = 0.5 * wco_ref[...]

    w2 = jnp.concatenate([wx, wh], axis=0)

    g = jnp.dot(x_ref[0], wx, preferred_element_type=jnp.float32)
    gate_i = _sigmoid_prescaled(g[:, 0:_H] + bias_i)
    cand = jnp.tanh(g[:, 2 * _H:3 * _H] + bias_c)
    c = gate_i * cand
    gate_o = _sigmoid_prescaled(g[:, 3 * _H:4 * _H] + (bias_o + wco * c))
    h = gate_o * jnp.tanh(c)

    for t in range(1, _T):
        xh = jnp.concatenate([x_ref[t], h], axis=1)
        g = jnp.dot(xh, w2, preferred_element_type=jnp.float32)
        gate_i = _sigmoid_prescaled(g[:, 0:_H] + (bias_i + wci * c))
        gate_f = _sigmoid_prescaled(g[:, _H:2 * _H] + (bias_f + wcf * c))
        cand = jnp.tanh(g[:, 2 * _H:3 * _H] + bias_c)
        c = gate_f * c + gate_i * cand
        gate_o = _sigmoid_prescaled(g[:, 3 * _H:4 * _H] + (bias_o + wco * c))
        h = gate_o * jnp.tanh(c)

    h_ref[...] = h
    c_ref[...] = c
    out_ref[...] = (jnp.dot(h, wout_ref[...], preferred_element_type=jnp.float32)
                    + bout_ref[...])


def kernel(x, edge_index, edge_weight, Wl, bl, Wr, br, att, gat_bias,
           Wxi, bxi, Whi, bhi, Wxf, bxf, Whf, bhf, Wxc, bxc, Whc, bhc,
           Wxo, bxo, Who, bho, wci, wcf, wco, bi, bf, bc, bo, Wout, bout):
    t_win, n, f = x.shape
    assert t_win == _T and f == _H
    rows = _ROWS
    grid = n // rows

    mat = pl.BlockSpec((_H, _H), lambda i: (0, 0))
    vec = pl.BlockSpec((_H,), lambda i: (0,))

    out2d, h, c = pl.pallas_call(
        _lstm_block_kernel,
        grid=(grid,),
        in_specs=[
            pl.BlockSpec((_T, rows, _H), lambda i: (0, i, 0)),
            mat, mat, mat, mat, mat, mat, mat, mat,
            vec, vec, vec, vec, vec, vec, vec, vec, vec, vec, vec, vec,
            vec, vec, vec,
            pl.BlockSpec((_H, 1), lambda i: (0, 0)),
            pl.BlockSpec((1,), lambda i: (0,)),
        ],
        out_specs=[
            pl.BlockSpec((rows, 1), lambda i: (i, 0)),
            pl.BlockSpec((rows, _H), lambda i: (i, 0)),
            pl.BlockSpec((rows, _H), lambda i: (i, 0)),
        ],
        out_shape=[
            jax.ShapeDtypeStruct((n, 1), jnp.float32),
            jax.ShapeDtypeStruct((n, _H), jnp.float32),
            jax.ShapeDtypeStruct((n, _H), jnp.float32),
        ],
        compiler_params=pltpu.CompilerParams(
            dimension_semantics=("parallel",)),
    )(x, Wxi, Whi, Wxf, Whf, Wxc, Whc, Wxo, Who,
      bxi, bhi, bi, bxf, bhf, bf, bxc, bhc, bc, bxo, bho, bo,
      wci, wcf, wco, Wout, bout)

    return (out2d[:, 0], (h, c))
